# baseline (speedup 1.0000x reference)
import functools

import jax
import jax.numpy as jnp
from jax import lax
from jax.experimental import pallas as pl
from jax.experimental.pallas import tpu as pltpu
from jax.experimental.pallas import tpu_sc as plsc

LANES = 128
BM = 2048
NE = 8
SCL = 16


def _tc_stage_kernel(env_ref, ep_ref, wcat_ref, b1_ref, sel_ref,
                     wg2_ref, bg2_ref, wg3_ref, bg3_ref, out_ref):
    acc = jnp.dot(env_ref[:], wcat_ref[:], preferred_element_type=jnp.float32)
    acc = acc + jnp.dot(ep_ref[:], sel_ref[:], preferred_element_type=jnp.float32)
    acc = acc + b1_ref[:]

    h1 = jnp.maximum(acc, 0.0)
    h2 = jnp.maximum(
        jnp.dot(h1, wg2_ref[:], preferred_element_type=jnp.float32) + bg2_ref[:], 0.0)
    logits = jnp.dot(h2, wg3_ref[:], preferred_element_type=jnp.float32) + bg3_ref[:]

    bm = logits.shape[0]
    lane = jax.lax.broadcasted_iota(jnp.int32, (bm, LANES), 1)
    gate_lane = jnp.logical_and(lane >= NE, lane < 2 * NE)
    lm = jnp.where(gate_lane, logits, jnp.float32(-1e30))
    mx = jnp.max(lm, axis=1, keepdims=True)
    ex = jnp.exp(lm - mx)
    g = ex / jnp.sum(ex, axis=1, keepdims=True)

    comb = jnp.where(lane < NE, acc, g)
    out_ref[:] = comb[:, 0:2 * NE]


def _tc_stage(env, ep, wcat, b1, sel, wg2p, bg2p, wg3p, bg3p):
    B, D = env.shape
    grid = (B // BM,)
    return pl.pallas_call(
        _tc_stage_kernel,
        grid=grid,
        in_specs=[
            pl.BlockSpec((BM, D), lambda i: (i, 0)),
            pl.BlockSpec((BM, 8), lambda i: (i, 0)),
            pl.BlockSpec((D, LANES), lambda i: (0, 0)),
            pl.BlockSpec((1, LANES), lambda i: (0, 0)),
            pl.BlockSpec((8, LANES), lambda i: (0, 0)),
            pl.BlockSpec((LANES, LANES), lambda i: (0, 0)),
            pl.BlockSpec((1, LANES), lambda i: (0, 0)),
            pl.BlockSpec((LANES, LANES), lambda i: (0, 0)),
            pl.BlockSpec((1, LANES), lambda i: (0, 0)),
        ],
        out_specs=pl.BlockSpec((BM, 2 * NE), lambda i: (i, 0)),
        out_shape=jax.ShapeDtypeStruct((B, 2 * NE), jnp.float32),
    )(env, ep, wcat, b1, sel, wg2p, bg2p, wg3p, bg3p)


def _sc_route(comb):
    B = comb.shape[0] // (2 * NE)
    info = plsc.get_sparse_core_info()
    nw = info.num_cores * info.num_subcores
    bw = B // nw
    mesh = plsc.VectorSubcoreMesh(core_axis_name="c", subcore_axis_name="s")

    @functools.partial(
        pl.kernel, mesh=mesh,
        out_type=jax.ShapeDtypeStruct((B,), jnp.float32),
        scratch_types=[
            pltpu.VMEM((bw * 2 * NE,), jnp.float32),
            pltpu.VMEM((bw,), jnp.float32),
        ],
        compiler_params=pltpu.CompilerParams(needs_layout_passes=False),
    )
    def k(comb_hbm, out_hbm, loc, res):
        wid = lax.axis_index("s") * info.num_cores + lax.axis_index("c")
        base = wid * bw
        pltpu.sync_copy(comb_hbm.at[pl.ds(base * 2 * NE, bw * 2 * NE)], loc)

        def body(t, _):
            tok = (lax.iota(jnp.int32, SCL) + t * SCL) * (2 * NE)
            eo = [plsc.load_gather(loc, [tok + e]) for e in range(NE)]
            g = [plsc.load_gather(loc, [tok + NE + e]) for e in range(NE)]
            one = jnp.ones((SCL,), jnp.float32)
            zero = jnp.zeros((SCL,), jnp.float32)
            rank = [zero] * NE
            for i in range(NE):
                for j in range(i + 1, NE):
                    j_gt = g[j] > g[i]
                    rank[i] = rank[i] + jnp.where(j_gt, one, zero)
                    rank[j] = rank[j] + jnp.where(j_gt, zero, one)
            num = zero
            den = zero
            for e in range(NE):
                w = jnp.where(rank[e] < 3.0, jnp.exp(g[e]), zero)
                num = num + w * eo[e]
                den = den + w
            res[pl.ds(t * SCL, SCL)] = num / den
            return 0

        lax.fori_loop(0, bw // SCL, body, 0)
        pltpu.sync_copy(res, out_hbm.at[pl.ds(base, bw)])

    return k(comb)


def kernel(env, experts_predictions, We0, be0, We1, be1, We2, be2, We3, be3,
           We4, be4, Wg1, bg1, Wg2, bg2, Wg3, bg3):
    D = env.shape[1]
    H = Wg1.shape[1]
    f32 = jnp.float32

    wcat = jnp.concatenate([
        We0, We1, We2, We3, We4[:, 0:1],
        jnp.zeros((D, 11), f32), Wg1,
        jnp.zeros((D, LANES - 16 - H), f32)], axis=1)
    b1 = jnp.concatenate([
        be0, be1, be2, be3, be4[0:1],
        jnp.zeros((11,), f32), bg1,
        jnp.zeros((LANES - 16 - H,), f32)]).reshape(1, LANES)
    sel = jnp.zeros((8, LANES), f32)
    sel = sel.at[0, 5].set(1.0).at[1, 6].set(1.0).at[2, 7].set(1.0)
    wg2p = jnp.zeros((LANES, LANES), f32).at[16:16 + H, 0:H].set(Wg2)
    bg2p = jnp.zeros((1, LANES), f32).at[0, 0:H].set(bg2)
    wg3p = jnp.zeros((LANES, LANES), f32).at[0:H, NE:2 * NE].set(Wg3)
    bg3p = jnp.zeros((1, LANES), f32).at[0, NE:2 * NE].set(bg3)
    ep = jnp.pad(experts_predictions, ((0, 0), (0, 5)))

    comb = _tc_stage(env, ep, wcat, b1, sel, wg2p, bg2p, wg3p, bg3p)
    return _sc_route(comb.reshape(-1))


# speedup vs baseline: 1.0292x; 1.0292x over previous
import functools

import jax
import jax.numpy as jnp
from jax import lax
from jax.experimental import pallas as pl
from jax.experimental.pallas import tpu as pltpu
from jax.experimental.pallas import tpu_sc as plsc

LANES = 128
BM = 2048
NE = 8
SCL = 16


def _tc_stage_kernel(env_ref, ep_ref, wcat_ref, b1_ref, sel_ref,
                     wg2_ref, bg2_ref, wg3_ref, bg3_ref, out_ref):
    acc = jnp.dot(env_ref[:], wcat_ref[:], preferred_element_type=jnp.float32)
    acc = acc + jnp.dot(ep_ref[:], sel_ref[:], preferred_element_type=jnp.float32)
    acc = acc + b1_ref[:]

    h1 = jnp.maximum(acc, 0.0)
    h2 = jnp.maximum(
        jnp.dot(h1, wg2_ref[:], preferred_element_type=jnp.float32) + bg2_ref[:], 0.0)
    logits = jnp.dot(h2, wg3_ref[:], preferred_element_type=jnp.float32) + bg3_ref[:]

    bm = logits.shape[0]
    lane = jax.lax.broadcasted_iota(jnp.int32, (bm, LANES), 1)
    gate_lane = jnp.logical_and(lane >= NE, lane < 2 * NE)
    lm = jnp.where(gate_lane, logits, jnp.float32(-1e30))
    mx = jnp.max(lm, axis=1, keepdims=True)
    ex = jnp.exp(lm - mx)
    g = ex / jnp.sum(ex, axis=1, keepdims=True)

    comb = jnp.where(lane < NE, acc, g)
    out_ref[:] = comb[:, 0:2 * NE]


def _tc_stage(env, ep, wcat, b1, sel, wg2p, bg2p, wg3p, bg3p, off, nb):
    B, D = env.shape
    grid = (nb,)
    return pl.pallas_call(
        _tc_stage_kernel,
        grid=grid,
        in_specs=[
            pl.BlockSpec((BM, D), lambda i: (i + off, 0)),
            pl.BlockSpec((BM, 8), lambda i: (i + off, 0)),
            pl.BlockSpec((D, LANES), lambda i: (0, 0)),
            pl.BlockSpec((1, LANES), lambda i: (0, 0)),
            pl.BlockSpec((8, LANES), lambda i: (0, 0)),
            pl.BlockSpec((LANES, LANES), lambda i: (0, 0)),
            pl.BlockSpec((1, LANES), lambda i: (0, 0)),
            pl.BlockSpec((LANES, LANES), lambda i: (0, 0)),
            pl.BlockSpec((1, LANES), lambda i: (0, 0)),
        ],
        out_specs=pl.BlockSpec((BM, 2 * NE), lambda i: (i, 0)),
        out_shape=jax.ShapeDtypeStruct((nb * BM, 2 * NE), jnp.float32),
    )(env, ep, wcat, b1, sel, wg2p, bg2p, wg3p, bg3p)


def _sc_route(comb):
    B = comb.shape[0] // (2 * NE)
    info = plsc.get_sparse_core_info()
    nw = info.num_cores * info.num_subcores
    bw = B // nw
    mesh = plsc.VectorSubcoreMesh(core_axis_name="c", subcore_axis_name="s")

    @functools.partial(
        pl.kernel, mesh=mesh,
        out_type=jax.ShapeDtypeStruct((B,), jnp.float32),
        scratch_types=[
            pltpu.VMEM((bw * 2 * NE,), jnp.float32),
            pltpu.VMEM((bw,), jnp.float32),
        ],
        compiler_params=pltpu.CompilerParams(needs_layout_passes=False),
    )
    def k(comb_hbm, out_hbm, loc, res):
        wid = lax.axis_index("s") * info.num_cores + lax.axis_index("c")
        base = wid * bw
        pltpu.sync_copy(comb_hbm.at[pl.ds(base * 2 * NE, bw * 2 * NE)], loc)

        def body(t, _):
            tok = (lax.iota(jnp.int32, SCL) + t * SCL) * (2 * NE)
            eo = [plsc.load_gather(loc, [tok + e]) for e in range(NE)]
            g = [plsc.load_gather(loc, [tok + NE + e]) for e in range(NE)]
            one = jnp.ones((SCL,), jnp.float32)
            zero = jnp.zeros((SCL,), jnp.float32)
            rank = [zero] * NE
            for i in range(NE):
                for j in range(i + 1, NE):
                    j_gt = g[j] > g[i]
                    rank[i] = rank[i] + jnp.where(j_gt, one, zero)
                    rank[j] = rank[j] + jnp.where(j_gt, zero, one)
            num = zero
            den = zero
            for e in range(NE):
                w = jnp.where(rank[e] < 3.0, jnp.exp(g[e]), zero)
                num = num + w * eo[e]
                den = den + w
            res[pl.ds(t * SCL, SCL)] = num / den
            return 0

        lax.fori_loop(0, bw // SCL, body, 0)
        pltpu.sync_copy(res, out_hbm.at[pl.ds(base, bw)])

    return k(comb)


def kernel(env, experts_predictions, We0, be0, We1, be1, We2, be2, We3, be3,
           We4, be4, Wg1, bg1, Wg2, bg2, Wg3, bg3):
    D = env.shape[1]
    H = Wg1.shape[1]
    f32 = jnp.float32

    wcat = jnp.concatenate([
        We0, We1, We2, We3, We4[:, 0:1],
        jnp.zeros((D, 11), f32), Wg1,
        jnp.zeros((D, LANES - 16 - H), f32)], axis=1)
    b1 = jnp.concatenate([
        be0, be1, be2, be3, be4[0:1],
        jnp.zeros((11,), f32), bg1,
        jnp.zeros((LANES - 16 - H,), f32)]).reshape(1, LANES)
    sel = jnp.zeros((8, LANES), f32)
    sel = sel.at[0, 5].set(1.0).at[1, 6].set(1.0).at[2, 7].set(1.0)
    wg2p = jnp.zeros((LANES, LANES), f32).at[16:16 + H, 0:H].set(Wg2)
    bg2p = jnp.zeros((1, LANES), f32).at[0, 0:H].set(bg2)
    wg3p = jnp.zeros((LANES, LANES), f32).at[0:H, NE:2 * NE].set(Wg3)
    bg3p = jnp.zeros((1, LANES), f32).at[0, NE:2 * NE].set(bg3)
    ep = jnp.pad(experts_predictions, ((0, 0), (0, 5)))

    B = env.shape[0]
    nchunk = 2
    nb = B // BM // nchunk
    outs = []
    for c in range(nchunk):
        comb = _tc_stage(env, ep, wcat, b1, sel, wg2p, bg2p, wg3p, bg3p,
                         c * nb, nb)
        outs.append(_sc_route(comb.reshape(-1)))
    return jnp.concatenate(outs)
